# baseline (speedup 1.0000x reference)
import functools

import jax
import jax.numpy as jnp
from jax import lax
from jax.experimental import pallas as pl
from jax.experimental.pallas import tpu as pltpu
from jax.experimental.pallas import tpu_sc as plsc

_ROWS = 224 * 224
_COLS = 768
_NC = 2
_NS = 16
_NW = _NC * _NS
_WROWS = _ROWS // _NW
_CHUNK = 56
_NCHUNKS = _WROWS // _CHUNK


@functools.partial(
    pl.kernel,
    mesh=plsc.VectorSubcoreMesh(core_axis_name="c", subcore_axis_name="s"),
    out_type=jax.ShapeDtypeStruct((_ROWS, _COLS), jnp.float32),
    scratch_types=[pltpu.VMEM((_CHUNK, _COLS), jnp.float32)],
)
def _sc_copy(x_hbm, o_hbm, buf):
    wid = lax.axis_index("s") * _NC + lax.axis_index("c")
    base = wid * _WROWS
    for j in range(_NCHUNKS):
        pltpu.sync_copy(x_hbm.at[pl.ds(base + j * _CHUNK, _CHUNK)], buf)
        pltpu.sync_copy(buf, o_hbm.at[pl.ds(base + j * _CHUNK, _CHUNK)])


def kernel(input):
    x = input.reshape(_COLS, _ROWS).T
    out = _sc_copy(x)
    return out.T.reshape(input.shape)


# speedup vs baseline: 1.1332x; 1.1332x over previous
import functools

import jax
import jax.numpy as jnp
from jax import lax
from jax.experimental import pallas as pl
from jax.experimental.pallas import tpu as pltpu
from jax.experimental.pallas import tpu_sc as plsc

_ROWS = 224 * 224
_COLS = 768
_NC = 2
_NS = 16
_NW = _NC * _NS
_WROWS = _ROWS // _NW
_CHUNK = 32
_NCH = _WROWS // _CHUNK
_NBUF = 4
_LAG = 2


@functools.partial(
    pl.kernel,
    mesh=plsc.VectorSubcoreMesh(core_axis_name="c", subcore_axis_name="s"),
    out_type=jax.ShapeDtypeStruct((_ROWS, _COLS), jnp.float32),
    scratch_types=[pltpu.VMEM((_NBUF, _CHUNK, _COLS), jnp.float32)]
    + [pltpu.SemaphoreType.DMA] * (2 * _NBUF),
)
def _sc_copy(x_hbm, o_hbm, buf, *sems):
    rsems, wsems = sems[:_NBUF], sems[_NBUF:]
    wid = lax.axis_index("s") * _NC + lax.axis_index("c")
    base = wid * _WROWS

    def src(j):
        return x_hbm.at[pl.ds(base + j * _CHUNK, _CHUNK)]

    def dst(j):
        return o_hbm.at[pl.ds(base + j * _CHUNK, _CHUNK)]

    def issue_read(j):
        pltpu.async_copy(src(j), buf.at[j % _NBUF], rsems[j % _NBUF])

    def wait_read(j):
        pltpu.make_async_copy(src(j), buf.at[j % _NBUF], rsems[j % _NBUF]).wait()

    def issue_write(j):
        pltpu.async_copy(buf.at[j % _NBUF], dst(j), wsems[j % _NBUF])

    def wait_write(j):
        pltpu.make_async_copy(buf.at[j % _NBUF], dst(j), wsems[j % _NBUF]).wait()

    for j in range(_LAG):
        issue_read(j)
    for j in range(_NCH):
        wait_read(j)
        issue_write(j)
        nj = j + _LAG
        if nj < _NCH:
            if nj >= _NBUF:
                wait_write(nj - _NBUF)
            issue_read(nj)
    for j in range(_NCH - _NBUF, _NCH):
        wait_write(j)


def kernel(input):
    x = input.reshape(_COLS, _ROWS).T
    out = _sc_copy(x)
    return out.T.reshape(input.shape)


# speedup vs baseline: 1.1391x; 1.0052x over previous
import functools

import jax
import jax.numpy as jnp
from jax import lax
from jax.experimental import pallas as pl
from jax.experimental.pallas import tpu as pltpu
from jax.experimental.pallas import tpu_sc as plsc

_ROWS = 224 * 224
_COLS = 768
_NC = 2
_NS = 16
_NW = _NC * _NS
_WROWS = _ROWS // _NW
_CHUNK = 56
_NCH = _WROWS // _CHUNK
_NBUF = 2
_LAG = 1


@functools.partial(
    pl.kernel,
    mesh=plsc.VectorSubcoreMesh(core_axis_name="c", subcore_axis_name="s"),
    out_type=jax.ShapeDtypeStruct((_ROWS, _COLS), jnp.float32),
    scratch_types=[pltpu.VMEM((_NBUF, _CHUNK, _COLS), jnp.float32)]
    + [pltpu.SemaphoreType.DMA] * (2 * _NBUF),
)
def _sc_copy(x_hbm, o_hbm, buf, *sems):
    rsems, wsems = sems[:_NBUF], sems[_NBUF:]
    wid = lax.axis_index("s") * _NC + lax.axis_index("c")
    base = wid * _WROWS

    def src(j):
        return x_hbm.at[pl.ds(base + j * _CHUNK, _CHUNK)]

    def dst(j):
        return o_hbm.at[pl.ds(base + j * _CHUNK, _CHUNK)]

    def issue_read(j):
        pltpu.async_copy(src(j), buf.at[j % _NBUF], rsems[j % _NBUF])

    def wait_read(j):
        pltpu.make_async_copy(src(j), buf.at[j % _NBUF], rsems[j % _NBUF]).wait()

    def issue_write(j):
        pltpu.async_copy(buf.at[j % _NBUF], dst(j), wsems[j % _NBUF])

    def wait_write(j):
        pltpu.make_async_copy(buf.at[j % _NBUF], dst(j), wsems[j % _NBUF]).wait()

    for j in range(_LAG):
        issue_read(j)
    for j in range(_NCH):
        wait_read(j)
        issue_write(j)
        nj = j + _LAG
        if nj < _NCH:
            if nj >= _NBUF:
                wait_write(nj - _NBUF)
            issue_read(nj)
    for j in range(_NCH - _NBUF, _NCH):
        wait_write(j)


def kernel(input):
    x = input.reshape(_COLS, _ROWS).T
    out = _sc_copy(x)
    return out.T.reshape(input.shape)


# speedup vs baseline: 1.5183x; 1.3329x over previous
import jax
from jax.experimental import pallas as pl
from jax.experimental.pallas import tpu as pltpu

_ROWS = 224 * 224
_COLS = 768
_BLOCK_ROWS = 3584


def _copy_block(x_ref, o_ref):
    o_ref[...] = x_ref[...]


def kernel(input):
    x = input.reshape(_COLS, _ROWS).T
    out = pl.pallas_call(
        _copy_block,
        grid=(_ROWS // _BLOCK_ROWS,),
        in_specs=[pl.BlockSpec((_BLOCK_ROWS, _COLS), lambda i: (i, 0))],
        out_specs=pl.BlockSpec((_BLOCK_ROWS, _COLS), lambda i: (i, 0)),
        out_shape=jax.ShapeDtypeStruct((_ROWS, _COLS), x.dtype),
    )(x)
    return out.T.reshape(input.shape)
